# Initial kernel scaffold; baseline (speedup 1.0000x reference)
import functools
import jax
import jax.numpy as jnp
from jax import lax
from jax.experimental import pallas as pl
from jax.experimental.pallas import tpu as pltpu
from jax.experimental.pallas import tpu_sc as plsc

_VOCAB = 1000
_DIM = 64
_B = 4096
_X_MAX = 100.0
_ALPHA = 0.75

_NC = 2
_NS = 16
_NW = _NC * _NS
_CHUNK = _B // _NW
_L = 16


def _sc_gather_fn():
    mesh = plsc.VectorSubcoreMesh(core_axis_name="c", subcore_axis_name="s")

    @functools.partial(
        pl.kernel,
        mesh=mesh,
        out_type=(
            jax.ShapeDtypeStruct((_B,), jnp.float32),
            jax.ShapeDtypeStruct((_B,), jnp.float32),
            jax.ShapeDtypeStruct((_B,), jnp.float32),
        ),
        scratch_types=[
            pltpu.VMEM((_CHUNK,), jnp.int32),
            pltpu.VMEM((_CHUNK,), jnp.int32),
            pltpu.VMEM((_CHUNK,), jnp.int32),
            pltpu.VMEM((_CHUNK, _DIM), jnp.float32),
            pltpu.VMEM((_CHUNK, _DIM), jnp.float32),
            pltpu.VMEM((_CHUNK,), jnp.float32),
            pltpu.VMEM((_CHUNK,), jnp.float32),
            pltpu.VMEM((_CHUNK,), jnp.float32),
            pltpu.VMEM((_CHUNK,), jnp.float32),
            pltpu.VMEM((_CHUNK,), jnp.float32),
            pltpu.SemaphoreType.DMA,
        ],
    )
    def sc_kernel(words_hbm, targets_hbm, w_hbm, tw_hbm, b_hbm, tb_hbm,
                  co_hbm, co_out, dot_out, c_out,
                  wv, tv, ci, ew, etw, bg, tbg, cov, dotv, cv, sem):
        wid = lax.axis_index("s") * _NC + lax.axis_index("c")
        base = wid * _CHUNK
        pltpu.sync_copy(words_hbm.at[pl.ds(base, _CHUNK)], wv)
        pltpu.sync_copy(targets_hbm.at[pl.ds(base, _CHUNK)], tv)
        for i in range(_CHUNK // _L):
            s = pl.ds(i * _L, _L)
            ci[s] = wv[s] * _VOCAB + tv[s]
        copies = [
            pltpu.async_copy(w_hbm.at[wv], ew, sem),
            pltpu.async_copy(tw_hbm.at[tv], etw, sem),
            pltpu.async_copy(b_hbm.at[wv], bg, sem),
            pltpu.async_copy(tb_hbm.at[tv], tbg, sem),
            pltpu.async_copy(co_hbm.at[ci], cov, sem),
        ]
        for cp in copies:
            cp.wait()

        def dot_body(k, carry):
            acc = ew[k, pl.ds(0, _L)] * etw[k, pl.ds(0, _L)]
            for j in range(1, _DIM // _L):
                s = pl.ds(j * _L, _L)
                acc = acc + ew[k, s] * etw[k, s]
            dotv[k] = jnp.sum(acc)
            return carry

        lax.fori_loop(0, _CHUNK, dot_body, 0)
        for i in range(_CHUNK // _L):
            s = pl.ds(i * _L, _L)
            cv[s] = bg[s] + tbg[s]
        pltpu.sync_copy(cov, co_out.at[pl.ds(base, _CHUNK)])
        pltpu.sync_copy(dotv, dot_out.at[pl.ds(base, _CHUNK)])
        pltpu.sync_copy(cv, c_out.at[pl.ds(base, _CHUNK)])

    return sc_kernel


def _tc_loss(co2, dot2, c2):
    def body(co_ref, dot_ref, c_ref, out_ref):
        co = co_ref[...]
        w = jnp.minimum(jnp.exp(jnp.log(co / _X_MAX) * _ALPHA), 1.0)
        a = w * dot_ref[...] - jnp.log(co)
        c = c_ref[...]
        s1 = jnp.sum(a * a)
        s2 = jnp.sum(a * w)
        s3 = jnp.sum(w * w)
        s4 = jnp.sum(c)
        s5 = jnp.sum(c * c)
        out_ref[0, 0] = _B * s1 + 2.0 * s2 * s4 + s3 * s5

    return pl.pallas_call(
        body,
        out_shape=jax.ShapeDtypeStruct((1, 1), jnp.float32),
        out_specs=pl.BlockSpec(memory_space=pltpu.SMEM),
    )(co2, dot2, c2)


def kernel(words, target_words, W, b, tilde_W, tilde_b, co_mat):
    co_flat = co_mat.reshape(-1)
    b_flat = b.reshape(-1)
    tb_flat = tilde_b.reshape(-1)
    co, dot, c = _sc_gather_fn()(
        words, target_words, W, tilde_W, b_flat, tb_flat, co_flat)
    loss = _tc_loss(co.reshape(_NW, _CHUNK), dot.reshape(_NW, _CHUNK),
                    c.reshape(_NW, _CHUNK))
    return loss[0, 0]


# speedup vs baseline: 2.7405x; 2.7405x over previous
import functools
import jax
import jax.numpy as jnp
from jax import lax
from jax.experimental import pallas as pl
from jax.experimental.pallas import tpu as pltpu
from jax.experimental.pallas import tpu_sc as plsc

_VOCAB = 1000
_DIM = 64
_B = 4096
_X_MAX = 100.0
_ALPHA = 0.75

_NC = 2
_NS = 16
_NW = _NC * _NS
_CHUNK = _B // _NW
_L = 16
_PW = _DIM + _L


def _sc_gather_fn():
    mesh = plsc.VectorSubcoreMesh(core_axis_name="c", subcore_axis_name="s")

    @functools.partial(
        pl.kernel,
        mesh=mesh,
        out_type=(
            jax.ShapeDtypeStruct((_B,), jnp.float32),
            jax.ShapeDtypeStruct((_B, _PW), jnp.float32),
        ),
        scratch_types=[
            pltpu.VMEM((_CHUNK,), jnp.int32),
            pltpu.VMEM((_CHUNK,), jnp.int32),
            pltpu.VMEM((_CHUNK,), jnp.int32),
            pltpu.VMEM((_CHUNK, 2 * _DIM), jnp.float32),
            pltpu.VMEM((_CHUNK, 2 * _DIM), jnp.float32),
            pltpu.VMEM((_CHUNK,), jnp.float32),
            pltpu.VMEM((_CHUNK, _PW), jnp.float32),
            pltpu.SemaphoreType.DMA,
        ],
    )
    def sc_kernel(words_hbm, targets_hbm, t1_hbm, t2_hbm, co_hbm,
                  co_out, p_out,
                  wv, tv, ci, ew, etw, cov, pv, sem):
        wid = lax.axis_index("s") * _NC + lax.axis_index("c")
        base = wid * _CHUNK
        pltpu.sync_copy(words_hbm.at[pl.ds(base, _CHUNK)], wv)
        pltpu.sync_copy(targets_hbm.at[pl.ds(base, _CHUNK)], tv)
        for i in range(_CHUNK // _L):
            s = pl.ds(i * _L, _L)
            ci[s] = wv[s] * _VOCAB + tv[s]
        copies = [
            pltpu.async_copy(t1_hbm.at[wv], ew, sem),
            pltpu.async_copy(t2_hbm.at[tv], etw, sem),
            pltpu.async_copy(co_hbm.at[ci], cov, sem),
        ]
        for cp in copies:
            cp.wait()

        def prod_body(k, carry):
            for j in range(_DIM // _L):
                s = pl.ds(j * _L, _L)
                pv[k, s] = ew[k, s] * etw[k, s]
            s = pl.ds(_DIM, _L)
            pv[k, s] = ew[k, s] + etw[k, s]
            return carry

        lax.fori_loop(0, _CHUNK, prod_body, 0)
        pltpu.sync_copy(cov, co_out.at[pl.ds(base, _CHUNK)])
        pltpu.sync_copy(pv, p_out.at[pl.ds(base, _CHUNK), :])

    return sc_kernel


def _tc_loss(co2, p):
    def body(co_ref, p_ref, out_ref):
        co = co_ref[...]
        blk = p_ref[...]
        dot = jnp.sum(blk[:, :_DIM], axis=1, keepdims=True)
        c = jnp.sum(blk[:, _DIM:], axis=1, keepdims=True)
        w = jnp.minimum(jnp.exp(jnp.log(co / _X_MAX) * _ALPHA), 1.0)
        a = w * dot - jnp.log(co)
        s1 = jnp.sum(a * a)
        s2 = jnp.sum(a * w)
        s3 = jnp.sum(w * w)
        s4 = jnp.sum(c)
        s5 = jnp.sum(c * c)
        out_ref[0, 0] = _B * s1 + 2.0 * s2 * s4 + s3 * s5

    return pl.pallas_call(
        body,
        out_shape=jax.ShapeDtypeStruct((1, 1), jnp.float32),
        out_specs=pl.BlockSpec(memory_space=pltpu.SMEM),
    )(co2, p)


def kernel(words, target_words, W, b, tilde_W, tilde_b, co_mat):
    pad = jnp.zeros((_VOCAB, _DIM - 1), jnp.float32)
    t1 = jnp.concatenate([W, b, pad], axis=1)
    t2 = jnp.concatenate([tilde_W, tilde_b, pad], axis=1)
    co_flat = co_mat.reshape(-1)
    co, p = _sc_gather_fn()(words, target_words, t1, t2, co_flat)
    loss = _tc_loss(co.reshape(_B, 1), p)
    return loss[0, 0]


# speedup vs baseline: 3.2028x; 1.1687x over previous
import functools
import jax
import jax.numpy as jnp
from jax import lax
from jax.experimental import pallas as pl
from jax.experimental.pallas import tpu as pltpu
from jax.experimental.pallas import tpu_sc as plsc

_VOCAB = 1000
_DIM = 64
_B = 4096
_X_MAX = 100.0
_ALPHA = 0.75

_NC = 2
_NS = 16
_NW = _NC * _NS
_CHUNK = _B // _NW
_L = 16


def _sc_gather_fn():
    mesh = plsc.VectorSubcoreMesh(core_axis_name="c", subcore_axis_name="s")

    @functools.partial(
        pl.kernel,
        mesh=mesh,
        out_type=(
            jax.ShapeDtypeStruct((_B,), jnp.float32),
            jax.ShapeDtypeStruct((_B,), jnp.float32),
            jax.ShapeDtypeStruct((_B,), jnp.float32),
        ),
        scratch_types=[
            pltpu.VMEM((_CHUNK,), jnp.int32),
            pltpu.VMEM((_CHUNK,), jnp.int32),
            pltpu.VMEM((_CHUNK,), jnp.int32),
            pltpu.VMEM((_CHUNK, _DIM), jnp.float32),
            pltpu.VMEM((_CHUNK, _DIM), jnp.float32),
            pltpu.VMEM((_CHUNK,), jnp.float32),
            pltpu.VMEM((_CHUNK,), jnp.float32),
            pltpu.VMEM((_CHUNK,), jnp.float32),
            pltpu.VMEM((_CHUNK * _L,), jnp.float32),
            pltpu.VMEM((_CHUNK,), jnp.float32),
            pltpu.VMEM((_CHUNK,), jnp.float32),
            pltpu.SemaphoreType.DMA,
        ],
        compiler_params=pltpu.CompilerParams(
            use_tc_tiling_on_sc=False,
            needs_layout_passes=False,
        ),
    )
    def sc_kernel(words_hbm, targets_hbm, w_hbm, tw_hbm, b_hbm, tb_hbm,
                  co_hbm, co_out, dot_out, c_out,
                  wv, tv, ci, ew, etw, bg, tbg, cov, qflat, dotv, cv, sem):
        wid = lax.axis_index("s") * _NC + lax.axis_index("c")
        base = wid * _CHUNK
        pltpu.sync_copy(words_hbm.at[pl.ds(base, _CHUNK)], wv)
        pltpu.sync_copy(targets_hbm.at[pl.ds(base, _CHUNK)], tv)
        for i in range(_CHUNK // _L):
            s = pl.ds(i * _L, _L)
            ci[s] = wv[s] * _VOCAB + tv[s]
        copies = [
            pltpu.async_copy(w_hbm.at[wv], ew, sem),
            pltpu.async_copy(tw_hbm.at[tv], etw, sem),
            pltpu.async_copy(b_hbm.at[wv], bg, sem),
            pltpu.async_copy(tb_hbm.at[tv], tbg, sem),
            pltpu.async_copy(co_hbm.at[ci], cov, sem),
        ]
        for cp in copies:
            cp.wait()

        def fold_body(k, carry):
            s = pl.ds(0, _L)
            acc = ew[k, s] * etw[k, s]
            for j in range(1, _DIM // _L):
                s = pl.ds(j * _L, _L)
                acc = acc + ew[k, s] * etw[k, s]
            qflat[pl.ds(k * _L, _L)] = acc
            return carry

        lax.fori_loop(0, _CHUNK, fold_body, 0)

        lanebase = lax.iota(jnp.int32, _L) * _L
        for g in range(_CHUNK // _L):
            def red_body(d, acc, _g=g):
                return acc + plsc.load_gather(
                    qflat, [lanebase + (_g * _L * _L + d)])

            acc = lax.fori_loop(0, _L, red_body,
                                jnp.zeros((_L,), jnp.float32))
            dotv[pl.ds(g * _L, _L)] = acc

        for i in range(_CHUNK // _L):
            s = pl.ds(i * _L, _L)
            cv[s] = bg[s] + tbg[s]
        pltpu.sync_copy(cov, co_out.at[pl.ds(base, _CHUNK)])
        pltpu.sync_copy(dotv, dot_out.at[pl.ds(base, _CHUNK)])
        pltpu.sync_copy(cv, c_out.at[pl.ds(base, _CHUNK)])

    return sc_kernel


def _tc_loss(co2, dot2, c2):
    def body(co_ref, dot_ref, c_ref, out_ref):
        co = co_ref[...]
        dot = dot_ref[...]
        c = c_ref[...]
        w = jnp.minimum(jnp.exp(jnp.log(co / _X_MAX) * _ALPHA), 1.0)
        a = w * dot - jnp.log(co)
        s1 = jnp.sum(a * a)
        s2 = jnp.sum(a * w)
        s3 = jnp.sum(w * w)
        s4 = jnp.sum(c)
        s5 = jnp.sum(c * c)
        out_ref[0, 0] = _B * s1 + 2.0 * s2 * s4 + s3 * s5

    return pl.pallas_call(
        body,
        out_shape=jax.ShapeDtypeStruct((1, 1), jnp.float32),
        out_specs=pl.BlockSpec(memory_space=pltpu.SMEM),
    )(co2, dot2, c2)


def kernel(words, target_words, W, b, tilde_W, tilde_b, co_mat):
    co_flat = co_mat.reshape(-1)
    b_flat = b.reshape(-1)
    tb_flat = tilde_b.reshape(-1)
    co, dot, c = _sc_gather_fn()(
        words, target_words, W, tilde_W, b_flat, tb_flat, co_flat)
    loss = _tc_loss(co.reshape(_NW, _CHUNK), dot.reshape(_NW, _CHUNK),
                    c.reshape(_NW, _CHUNK))
    return loss[0, 0]
